# Initial kernel scaffold; baseline (speedup 1.0000x reference)
import jax
import jax.numpy as jnp
from jax.experimental import pallas as pl


def kernel(raw_keys, raw_values, map_table):
    raise NotImplementedError("write your pallas kernel here")


# speedup vs baseline: 5.8010x; 5.8010x over previous
import functools

import jax
import jax.numpy as jnp
from jax import lax
from jax.experimental import pallas as pl
from jax.experimental.pallas import tpu as pltpu
from jax.experimental.pallas import tpu_sc as plsc


def kernel(raw_keys, raw_values, map_table):
    B = raw_keys.shape[0]
    V = map_table.shape[0]

    info = plsc.get_sparse_core_info()
    NC, NS, L = info.num_cores, info.num_subcores, info.num_lanes
    NW = NC * NS
    b_per_w = B // NW

    V_pad = (V + 15) // 16 * 16
    table_padded = jnp.pad(map_table, (0, V_pad - V))

    mesh = plsc.VectorSubcoreMesh(core_axis_name="c", subcore_axis_name="s")

    @functools.partial(
        pl.kernel,
        mesh=mesh,
        compiler_params=pltpu.CompilerParams(needs_layout_passes=False),
        out_type=jax.ShapeDtypeStruct((B,), jnp.int32),
        scratch_types=[
            pltpu.VMEM((V_pad,), jnp.int32),
            pltpu.VMEM((b_per_w,), jnp.int32),
            pltpu.VMEM((b_per_w,), jnp.int32),
        ],
    )
    def lookup(keys_hbm, table_hbm, out_hbm, table_v, keys_v, out_v):
        wid = lax.axis_index("s") * NC + lax.axis_index("c")
        base = wid * b_per_w
        pltpu.sync_copy(table_hbm, table_v)
        pltpu.sync_copy(keys_hbm.at[pl.ds(base, b_per_w)], keys_v)
        for i in range(b_per_w // L):
            idx = keys_v[pl.ds(i * L, L)]
            out_v[pl.ds(i * L, L)] = plsc.load_gather(table_v, [idx])
        pltpu.sync_copy(out_v, out_hbm.at[pl.ds(base, b_per_w)])

    mapped = lookup(raw_keys, table_padded)
    return (mapped, raw_values)


# speedup vs baseline: 5.9568x; 1.0268x over previous
import functools

import jax
import jax.numpy as jnp
from jax import lax
from jax.experimental import pallas as pl
from jax.experimental.pallas import tpu as pltpu
from jax.experimental.pallas import tpu_sc as plsc


def kernel(raw_keys, raw_values, map_table):
    B = raw_keys.shape[0]
    V = map_table.shape[0]

    info = plsc.get_sparse_core_info()
    NC, NS, L = info.num_cores, info.num_subcores, info.num_lanes
    NW = NC * NS
    b_per_w = B // NW

    V_pad = (V + 15) // 16 * 16
    table_padded = jnp.pad(map_table, (0, V_pad - V))

    mesh = plsc.VectorSubcoreMesh(core_axis_name="c", subcore_axis_name="s")

    @functools.partial(
        pl.kernel,
        mesh=mesh,
        compiler_params=pltpu.CompilerParams(needs_layout_passes=False),
        out_type=jax.ShapeDtypeStruct((B,), jnp.int32),
        scratch_types=[
            pltpu.VMEM((V_pad,), jnp.int32),
            pltpu.VMEM((b_per_w,), jnp.int32),
            pltpu.VMEM((b_per_w,), jnp.int32),
        ],
    )
    def lookup(keys_hbm, table_hbm, out_hbm, table_v, keys_v, out_v):
        wid = lax.axis_index("s") * NC + lax.axis_index("c")
        base = wid * b_per_w
        pltpu.sync_copy(
            (table_hbm, keys_hbm.at[pl.ds(base, b_per_w)]),
            (table_v, keys_v),
        )
        for i in range(b_per_w // L):
            idx = keys_v[pl.ds(i * L, L)]
            out_v[pl.ds(i * L, L)] = plsc.load_gather(table_v, [idx])
        pltpu.sync_copy(out_v, out_hbm.at[pl.ds(base, b_per_w)])

    mapped = lookup(raw_keys, table_padded)
    return (mapped, raw_values)


# speedup vs baseline: 6.3688x; 1.0692x over previous
import functools

import jax
import jax.numpy as jnp
from jax import lax
from jax.experimental import pallas as pl
from jax.experimental.pallas import tpu as pltpu
from jax.experimental.pallas import tpu_sc as plsc


def kernel(raw_keys, raw_values, map_table):
    B = raw_keys.shape[0]
    V = map_table.shape[0]

    info = plsc.get_sparse_core_info()
    NC, NS, L = info.num_cores, info.num_subcores, info.num_lanes
    NW = NC * NS
    b_per_w = B // NW

    V_pad = (V + 15) // 16 * 16
    table_padded = jnp.pad(map_table, (0, V_pad - V))

    NC = 1
    NW = NC * NS
    b_per_w = B // NW
    mesh = plsc.VectorSubcoreMesh(
        core_axis_name="c", subcore_axis_name="s", num_cores=NC
    )

    @functools.partial(
        pl.kernel,
        mesh=mesh,
        compiler_params=pltpu.CompilerParams(needs_layout_passes=False),
        out_type=jax.ShapeDtypeStruct((B,), jnp.int32),
        scratch_types=[
            pltpu.VMEM((V_pad,), jnp.int32),
            pltpu.VMEM((b_per_w,), jnp.int32),
            pltpu.VMEM((b_per_w,), jnp.int32),
        ],
    )
    def lookup(keys_hbm, table_hbm, out_hbm, table_v, keys_v, out_v):
        wid = lax.axis_index("s") * NC + lax.axis_index("c")
        base = wid * b_per_w
        pltpu.sync_copy(
            (table_hbm, keys_hbm.at[pl.ds(base, b_per_w)]),
            (table_v, keys_v),
        )
        for i in range(b_per_w // L):
            idx = keys_v[pl.ds(i * L, L)]
            out_v[pl.ds(i * L, L)] = plsc.load_gather(table_v, [idx])
        pltpu.sync_copy(out_v, out_hbm.at[pl.ds(base, b_per_w)])

    mapped = lookup(raw_keys, table_padded)
    return (mapped, raw_values)


# speedup vs baseline: 6.4118x; 1.0068x over previous
import functools

import jax
import jax.numpy as jnp
from jax import lax
from jax.experimental import pallas as pl
from jax.experimental.pallas import tpu as pltpu
from jax.experimental.pallas import tpu_sc as plsc


def kernel(raw_keys, raw_values, map_table):
    B = raw_keys.shape[0]
    V = map_table.shape[0]

    info = plsc.get_sparse_core_info()
    NC, NS, L = info.num_cores, info.num_subcores, info.num_lanes
    NW = NC * NS
    b_per_w = B // NW

    V_pad = V

    NC = 1
    NW = NC * NS
    b_per_w = B // NW
    mesh = plsc.VectorSubcoreMesh(
        core_axis_name="c", subcore_axis_name="s", num_cores=NC
    )

    @functools.partial(
        pl.kernel,
        mesh=mesh,
        compiler_params=pltpu.CompilerParams(needs_layout_passes=False),
        out_type=jax.ShapeDtypeStruct((B,), jnp.int32),
        scratch_types=[
            pltpu.VMEM((V_pad,), jnp.int32),
            pltpu.VMEM((b_per_w,), jnp.int32),
            pltpu.VMEM((b_per_w,), jnp.int32),
        ],
    )
    def lookup(keys_hbm, table_hbm, out_hbm, table_v, keys_v, out_v):
        wid = lax.axis_index("s") * NC + lax.axis_index("c")
        base = wid * b_per_w
        pltpu.sync_copy(
            (table_hbm, keys_hbm.at[pl.ds(base, b_per_w)]),
            (table_v, keys_v),
        )
        for i in range(b_per_w // L):
            idx = keys_v[pl.ds(i * L, L)]
            out_v[pl.ds(i * L, L)] = plsc.load_gather(table_v, [idx])
        pltpu.sync_copy(out_v, out_hbm.at[pl.ds(base, b_per_w)])

    mapped = lookup(raw_keys, map_table)
    return (mapped, raw_values)


# speedup vs baseline: 6.4298x; 1.0028x over previous
import functools

import jax
import jax.numpy as jnp
from jax import lax
from jax.experimental import pallas as pl
from jax.experimental.pallas import tpu as pltpu
from jax.experimental.pallas import tpu_sc as plsc


def kernel(raw_keys, raw_values, map_table):
    B = raw_keys.shape[0]
    V = map_table.shape[0]

    info = plsc.get_sparse_core_info()
    NC, NS, L = info.num_cores, info.num_subcores, info.num_lanes
    NW = NC * NS
    b_per_w = B // NW

    V_pad = V

    NC = 1
    NW = NC * NS
    b_per_w = B // NW
    mesh = plsc.VectorSubcoreMesh(
        core_axis_name="c", subcore_axis_name="s", num_cores=NC
    )

    @functools.partial(
        pl.kernel,
        mesh=mesh,
        compiler_params=pltpu.CompilerParams(needs_layout_passes=False),
        out_type=jax.ShapeDtypeStruct((B,), jnp.int32),
        scratch_types=[
            pltpu.VMEM((V_pad,), jnp.int32),
            pltpu.VMEM((b_per_w,), jnp.int32),
            pltpu.VMEM((b_per_w,), jnp.int32),
            pltpu.SemaphoreType.DMA,
            pltpu.SemaphoreType.DMA,
        ],
    )
    def lookup(keys_hbm, table_hbm, out_hbm, table_v, keys_v, out_v, sem_in, sem_out):
        wid = lax.axis_index("s") * NC + lax.axis_index("c")
        base = wid * b_per_w
        in_table = pltpu.make_async_copy(table_hbm, table_v, sem_in)
        in_keys = pltpu.make_async_copy(
            keys_hbm.at[pl.ds(base, b_per_w)], keys_v, sem_in
        )
        in_table.start()
        in_keys.start()
        in_table.wait()
        in_keys.wait()
        half = b_per_w // 2
        out_cps = []
        for c in range(2):
            for i in range(c * half // L, (c + 1) * half // L):
                idx = keys_v[pl.ds(i * L, L)]
                out_v[pl.ds(i * L, L)] = plsc.load_gather(table_v, [idx])
            cp = pltpu.make_async_copy(
                out_v.at[pl.ds(c * half, half)],
                out_hbm.at[pl.ds(base + c * half, half)],
                sem_out,
            )
            cp.start()
            out_cps.append(cp)
        for cp in out_cps:
            cp.wait()

    mapped = lookup(raw_keys, map_table)
    return (mapped, raw_values)


# speedup vs baseline: 6.5019x; 1.0112x over previous
import functools

import jax
import jax.numpy as jnp
from jax import lax
from jax.experimental import pallas as pl
from jax.experimental.pallas import tpu as pltpu
from jax.experimental.pallas import tpu_sc as plsc


def kernel(raw_keys, raw_values, map_table):
    B = raw_keys.shape[0]
    V = map_table.shape[0]

    info = plsc.get_sparse_core_info()
    NC, NS, L = info.num_cores, info.num_subcores, info.num_lanes
    NW = NC * NS
    b_per_w = B // NW

    V_pad = V

    NC = 1
    NW = NC * NS
    b_per_w = B // NW
    mesh = plsc.VectorSubcoreMesh(
        core_axis_name="c", subcore_axis_name="s", num_cores=NC
    )

    @functools.partial(
        pl.kernel,
        mesh=mesh,
        compiler_params=pltpu.CompilerParams(needs_layout_passes=False),
        out_type=jax.ShapeDtypeStruct((B,), jnp.int32),
        scratch_types=[
            pltpu.VMEM((V_pad,), jnp.int32),
            pltpu.VMEM((b_per_w,), jnp.int32),
            pltpu.VMEM((b_per_w,), jnp.int32),
            pltpu.SemaphoreType.DMA,
            pltpu.SemaphoreType.DMA,
        ],
    )
    def lookup(keys_hbm, table_hbm, out_hbm, table_v, keys_v, out_v, sem_in, sem_out):
        wid = lax.axis_index("s") * NC + lax.axis_index("c")
        base = wid * b_per_w
        in_table = pltpu.make_async_copy(table_hbm, table_v, sem_in)
        in_keys = pltpu.make_async_copy(
            keys_hbm.at[pl.ds(base, b_per_w)], keys_v, sem_in
        )
        in_table.start()
        in_keys.start()
        in_table.wait()
        in_keys.wait()
        half = b_per_w // 2
        out_cps = []
        for c in range(2):
            @plsc.parallel_loop(c * half, (c + 1) * half, L, unroll=8)
            def _(i):
                idx = keys_v[pl.ds(i, L)]
                out_v[pl.ds(i, L)] = plsc.load_gather(table_v, [idx])
            cp = pltpu.make_async_copy(
                out_v.at[pl.ds(c * half, half)],
                out_hbm.at[pl.ds(base + c * half, half)],
                sem_out,
            )
            cp.start()
            out_cps.append(cp)
        for cp in out_cps:
            cp.wait()

    mapped = lookup(raw_keys, map_table)
    return (mapped, raw_values)


# speedup vs baseline: 6.5298x; 1.0043x over previous
import functools

import jax
import jax.numpy as jnp
from jax import lax
from jax.experimental import pallas as pl
from jax.experimental.pallas import tpu as pltpu
from jax.experimental.pallas import tpu_sc as plsc


def kernel(raw_keys, raw_values, map_table):
    B = raw_keys.shape[0]
    V = map_table.shape[0]

    info = plsc.get_sparse_core_info()
    NC, NS, L = info.num_cores, info.num_subcores, info.num_lanes
    NW = NC * NS
    b_per_w = B // NW

    V_pad = V

    NC = 1
    NW = NC * NS
    b_per_w = B // NW
    mesh = plsc.VectorSubcoreMesh(
        core_axis_name="c", subcore_axis_name="s", num_cores=NC
    )

    @functools.partial(
        pl.kernel,
        mesh=mesh,
        compiler_params=pltpu.CompilerParams(needs_layout_passes=False),
        out_type=jax.ShapeDtypeStruct((B,), jnp.int32),
        scratch_types=[
            pltpu.VMEM((V_pad,), jnp.int32),
            pltpu.VMEM((b_per_w,), jnp.int32),
            pltpu.VMEM((b_per_w,), jnp.int32),
            pltpu.SemaphoreType.DMA,
            pltpu.SemaphoreType.DMA,
        ],
    )
    def lookup(keys_hbm, table_hbm, out_hbm, table_v, keys_v, out_v, sem_in, sem_out):
        wid = lax.axis_index("s") * NC + lax.axis_index("c")
        base = wid * b_per_w
        in_table = pltpu.make_async_copy(table_hbm, table_v, sem_in)
        in_keys = pltpu.make_async_copy(
            keys_hbm.at[pl.ds(base, b_per_w)], keys_v, sem_in
        )
        in_table.start()
        in_keys.start()
        in_table.wait()
        in_keys.wait()
        @plsc.parallel_loop(0, b_per_w, L, unroll=8)
        def _(i):
            idx = keys_v[pl.ds(i, L)]
            out_v[pl.ds(i, L)] = plsc.load_gather(table_v, [idx])

        pltpu.make_async_copy(
            out_v, out_hbm.at[pl.ds(base, b_per_w)], sem_out
        ).start()
        pltpu.make_async_copy(
            out_v, out_hbm.at[pl.ds(base, b_per_w)], sem_out
        ).wait()

    mapped = lookup(raw_keys, map_table)
    return (mapped, raw_values)
